# Initial kernel scaffold; baseline (speedup 1.0000x reference)
import jax
import jax.numpy as jnp
from jax.experimental import pallas as pl


def kernel(x, W1, b1, gamma, beta, run_mean, run_var, W2, b2):
    raise NotImplementedError("write your pallas kernel here")


# speedup vs baseline: 2.4323x; 2.4323x over previous
import functools

import jax
import jax.numpy as jnp
from jax.experimental import pallas as pl

_EPS = 1e-5
_NEG = -1e30


def _gate_body(x_ref, w1_ref, b1_ref, g_ref, be_ref, rm_ref, rv_ref, w2_ref,
               b2_ref, gates_ref, idx_ref):
    x = x_ref[...]
    w1 = w1_ref[...]
    h = jnp.dot(x[:, 0:256], w1[0:256, :], preferred_element_type=jnp.float32)
    for k0 in range(256, x.shape[1], 256):
        h = h + jnp.dot(x[:, k0:k0 + 256], w1[k0:k0 + 256, :],
                        preferred_element_type=jnp.float32)
    h = h + b1_ref[...]
    h = (h - rm_ref[...]) / jnp.sqrt(rv_ref[...] + _EPS) * g_ref[...] + be_ref[...]
    h = jnp.maximum(h, 0.0)
    logits = jnp.dot(h, w2_ref[...], preferred_element_type=jnp.float32) + b2_ref[...]

    n_e = logits.shape[1]
    iota_e = jax.lax.broadcasted_iota(jnp.int32, logits.shape, 1)
    max1 = jnp.max(logits, axis=1, keepdims=True)
    idx1 = jnp.min(jnp.where(logits == max1, iota_e, n_e), axis=1, keepdims=True)
    masked = jnp.where(iota_e == idx1, _NEG, logits)
    max2 = jnp.max(masked, axis=1, keepdims=True)
    idx2 = jnp.min(jnp.where(masked == max2, iota_e, n_e), axis=1, keepdims=True)

    e = jnp.exp(max2 - max1)
    denom = 1.0 + e
    g1 = 1.0 / denom
    g2 = e / denom

    gates = jnp.where(iota_e == idx1, g1, 0.0) + jnp.where(iota_e == idx2, g2, 0.0)
    gates_ref[...] = gates
    idx_ref[...] = jnp.concatenate([idx1, idx2], axis=1).astype(jnp.int32)


@jax.jit
def kernel(x, W1, b1, gamma, beta, run_mean, run_var, W2, b2):
    n, d = x.shape
    hidden = W1.shape[0]
    n_e = W2.shape[0]

    w1t = W1.T
    w2t = W2.T

    t = 512
    grid = (n // t,)
    vec = pl.BlockSpec((1, hidden), lambda i: (0, 0))
    gates, idx = pl.pallas_call(
        _gate_body,
        grid=grid,
        in_specs=[
            pl.BlockSpec((t, d), lambda i: (i, 0)),
            pl.BlockSpec((d, hidden), lambda i: (0, 0)),
            vec, vec, vec, vec, vec,
            pl.BlockSpec((hidden, n_e), lambda i: (0, 0)),
            pl.BlockSpec((1, n_e), lambda i: (0, 0)),
        ],
        out_specs=[
            pl.BlockSpec((t, n_e), lambda i: (i, 0)),
            pl.BlockSpec((t, 2), lambda i: (i, 0)),
        ],
        out_shape=[
            jax.ShapeDtypeStruct((n, n_e), jnp.float32),
            jax.ShapeDtypeStruct((n, 2), jnp.int32),
        ],
    )(x, w1t, b1[None, :], gamma[None, :], beta[None, :], run_mean[None, :],
      run_var[None, :], w2t, b2[None, :])
    return gates, idx
